# baseline (speedup 1.0000x reference)
import functools

import jax
import jax.numpy as jnp
from jax import lax
from jax.experimental import pallas as pl
from jax.experimental.pallas import tpu as pltpu
from jax.experimental.pallas import tpu_sc as plsc

PRE = 1000
IOU_T = 0.55
POST = 300
NSLOT = 1024
NCHUNK = NSLOT // 16
NC, NS = 2, 16
NW = NC * NS
NCLS = 80
QMAX = -(-NCLS // NW)
QN = QMAX * NSLOT
OUTN = 304
NEG = -3.0e38


def _wbf_sc_body(x_hbm, buf_hbm, xv, idxl, created, mcoord, wacc, sacc,
                 cacc, msv, arena, rowv, sem):
    wid = lax.axis_index("s") * NC + lax.axis_index("c")
    lanes = lax.broadcasted_iota(jnp.int32, (16,), 0)

    pltpu.sync_copy(x_hbm, xv)

    def part_q(q, carry):
        cfq = (wid + NW * q).astype(jnp.float32)
        qbase = q * NSLOT

        def part_body(ch, cnt):
            idxs = lanes + ch * 16
            cls_chunk = plsc.load_gather(xv, [idxs * 6 + 5])
            m = (cls_chunk == cfq) & (idxs < PRE)
            mi = jnp.where(m, 1, 0)
            pos = jnp.where(m, qbase + cnt + plsc.cumsum(mi) - 1, QN + lanes)
            plsc.store_scatter(idxl, [pos], idxs)
            return cnt + plsc.all_reduce_population_count(m)

        cnt = lax.fori_loop(0, NCHUNK, part_body, jnp.zeros((16,), jnp.int32))
        plsc.store_scatter(msv, [jnp.full((16,), q, jnp.int32)], cnt)
        return carry

    lax.fori_loop(0, QMAX, part_q, jnp.int32(0))

    def clus_q(q, carry):
        Q = q * NSLOT
        m_q = jnp.max(plsc.load_gather(msv, [jnp.full((16,), q, jnp.int32)]))

        def box_body(p, ncl):
            psp = jnp.full((16,), p, jnp.int32)
            isp = plsc.load_gather(idxl, [Q + psp])
            ia = isp * 6
            b0 = plsc.load_gather(xv, [ia])
            b1 = plsc.load_gather(xv, [ia + 1])
            b2 = plsc.load_gather(xv, [ia + 2])
            b3 = plsc.load_gather(xv, [ia + 3])
            s = plsc.load_gather(xv, [ia + 4])
            a1 = (b2 - b0) * (b3 - b1)

            def ch_body(ch, jf):
                base = ch * 16
                m0 = mcoord[pl.ds(0 * QN + Q + base, 16)]
                m1 = mcoord[pl.ds(1 * QN + Q + base, 16)]
                m2 = mcoord[pl.ds(2 * QN + Q + base, 16)]
                m3 = mcoord[pl.ds(3 * QN + Q + base, 16)]
                iw = jnp.maximum(jnp.minimum(b2, m2) - jnp.maximum(b0, m0), 0.0)
                ih = jnp.maximum(jnp.minimum(b3, m3) - jnp.maximum(b1, m1), 0.0)
                inter = iw * ih
                a2 = (m2 - m0) * (m3 - m1)
                iou = inter / (a1 + a2 - inter)
                hit = (iou > IOU_T) & ((lanes + base) < ncl)
                pc = plsc.all_reduce_population_count(hit)
                ff = plsc.all_reduce_ffs(hit)
                return jnp.where((jf >= 16384) & (pc > 0), base + ff, jf)

            nch = (ncl + 15) >> 4
            jf = lax.fori_loop(0, nch, ch_body,
                               jnp.full((16,), 16384, jnp.int32))
            anyv = jf < 16384
            anyh = jnp.max(jnp.where(anyv, 1, 0)) > 0
            ksp = jnp.where(anyv, jf, jnp.full((16,), ncl, jnp.int32))
            addr = Q + ksp
            old_ss = plsc.load_gather(sacc, [addr])
            ssc = jnp.where(anyv, old_ss + s, s)
            plsc.store_scatter(sacc, [addr], ssc)
            old_c = plsc.load_gather(cacc, [addr])
            plsc.store_scatter(cacc, [addr],
                               jnp.where(anyv, old_c + 1.0, 1.0))
            for d, bd in enumerate((b0, b1, b2, b3)):
                old_w = plsc.load_gather(wacc, [d * QN + addr])
                wc = jnp.where(anyv, old_w + s * bd, s * bd)
                plsc.store_scatter(wacc, [d * QN + addr], wc)
                plsc.store_scatter(mcoord, [d * QN + addr], wc / ssc)
            crv = jnp.where(anyv, jnp.full((16,), -1, jnp.int32),
                            jnp.full((16,), ncl, jnp.int32))
            plsc.store_scatter(created, [Q + psp], crv)
            return ncl + jnp.where(anyh, 0, 1)

        lax.fori_loop(0, m_q, box_body, jnp.int32(0))
        return carry

    lax.fori_loop(0, QMAX, clus_q, jnp.int32(0))

    def out_q(q, apos0):
        Q = q * NSLOT
        cfq = (wid + NW * q).astype(jnp.float32)
        m_q = jnp.max(plsc.load_gather(msv, [jnp.full((16,), q, jnp.int32)]))

        def out_body(p, apos):
            psp = jnp.full((16,), p, jnp.int32)
            isp = plsc.load_gather(idxl, [Q + psp])
            ksp = plsc.load_gather(created, [Q + psp])
            isnew = ksp >= 0
            addr = Q + jnp.maximum(ksp, 0)
            ssv = plsc.load_gather(sacc, [addr])
            cnv = plsc.load_gather(cacc, [addr])
            scv = ssv / cnv
            keyv = jnp.where(isnew, scv, jnp.full((16,), NEG, jnp.float32))
            tbv = cfq * 1024.0 + isp.astype(jnp.float32)
            ab = apos * 16
            absp = jnp.full((16,), ab, jnp.int32)
            for d in range(4):
                wv = plsc.load_gather(wacc, [d * QN + addr])
                plsc.store_scatter(arena, [absp + d], wv / ssv)
            plsc.store_scatter(arena, [absp + 4], scv)
            plsc.store_scatter(arena, [absp + 5],
                               jnp.full((16,), cfq, jnp.float32))
            plsc.store_scatter(arena, [absp + 6], keyv)
            plsc.store_scatter(arena, [absp + 7], tbv)
            i_sc = jnp.max(isp)
            pltpu.async_copy(arena.at[pl.ds(ab, 16)],
                             buf_hbm.at[pl.ds(i_sc * 16, 16)], sem)
            return apos + 1

        return lax.fori_loop(0, m_q, out_body, apos0)

    nfired = lax.fori_loop(0, QMAX, out_q, jnp.int32(0))

    def drain_body(p, c):
        pltpu.make_async_copy(x_hbm.at[pl.ds(0, 16)], rowv, sem).wait()
        return c

    lax.fori_loop(0, nfired, drain_body, jnp.int32(0))


def _rank_tc_body(buf_ref, o_ref):
    jcol = lax.broadcasted_iota(jnp.int32, (NSLOT, 1), 0)
    vcol = jcol < PRE
    key_c = jnp.where(vcol, buf_ref[:, 6:7], NEG)
    tb_c = jnp.where(vcol, buf_ref[:, 7:8], 2.0e8 + jcol.astype(jnp.float32))
    key_r = jnp.reshape(key_c, (1, NSLOT))
    tb_r = jnp.reshape(tb_c, (1, NSLOT))
    before = (key_c > key_r) | ((key_c == key_r) & (tb_c < tb_r))
    rank = jnp.sum(before.astype(jnp.float32), axis=0, keepdims=True)
    rsel = lax.broadcasted_iota(jnp.int32, (OUTN, 1), 0).astype(jnp.float32)
    onehot = (rank == rsel).astype(jnp.float32)
    bufc = jnp.where(vcol, buf_ref[:], 0.0)
    res = lax.dot_general(
        onehot, bufc, (((1,), (0,)), ((), ())),
        precision=lax.Precision.HIGHEST,
        preferred_element_type=jnp.float32)
    o_ref[:] = res[:POST, :6]


@jax.jit
def kernel(x):
    xflat = x.astype(jnp.float32).reshape(x.shape[0] * 6)

    mesh = plsc.VectorSubcoreMesh(core_axis_name="c", subcore_axis_name="s",
                                  num_cores=NC, num_subcores=NS)
    phase1 = pl.kernel(
        _wbf_sc_body,
        out_type=jax.ShapeDtypeStruct((NSLOT * 16,), jnp.float32),
        mesh=mesh,
        compiler_params=pltpu.CompilerParams(needs_layout_passes=False),
        scratch_types=[
            pltpu.VMEM((7200,), jnp.float32),
            pltpu.VMEM((QN + 16,), jnp.int32),
            pltpu.VMEM((QN,), jnp.int32),
            pltpu.VMEM((4 * QN,), jnp.float32),
            pltpu.VMEM((4 * QN,), jnp.float32),
            pltpu.VMEM((QN,), jnp.float32),
            pltpu.VMEM((QN,), jnp.float32),
            pltpu.VMEM((16,), jnp.int32),
            pltpu.VMEM((NSLOT * 16,), jnp.float32),
            pltpu.VMEM((16,), jnp.float32),
            pltpu.SemaphoreType.DMA,
        ],
    )
    buf = phase1(xflat).reshape(NSLOT, 16)

    return pl.pallas_call(
        _rank_tc_body,
        out_shape=jax.ShapeDtypeStruct((POST, 6), jnp.float32),
    )(buf)


# speedup vs baseline: 1.0210x; 1.0210x over previous
import functools

import jax
import jax.numpy as jnp
from jax import lax
from jax.experimental import pallas as pl
from jax.experimental.pallas import tpu as pltpu
from jax.experimental.pallas import tpu_sc as plsc

PRE = 1000
IOU_T = 0.55
POST = 300
NSLOT = 1024
NCHUNK = NSLOT // 16
NC, NS = 2, 16
NW = NC * NS
NCLS = 80
QMAX = -(-NCLS // NW)
QN = QMAX * NSLOT
OUTN = 304
NEG = -3.0e38


def _wbf_sc_body(x_hbm, buf_hbm, xv, idxl, created, mcoord, wacc, sacc,
                 cacc, arena, rowv, sem):
    wid = lax.axis_index("s") * NC + lax.axis_index("c")
    lanes = lax.broadcasted_iota(jnp.int32, (16,), 0)

    pltpu.sync_copy(x_hbm, xv)

    cfs = [(wid + NW * q).astype(jnp.float32) for q in range(QMAX)]

    def part_body(ch, cnts):
        idxs = lanes + ch * 16
        cls_chunk = plsc.load_gather(xv, [idxs * 6 + 5])
        live = idxs < PRE
        out = []
        for q in range(QMAX):
            m = (cls_chunk == cfs[q]) & live
            mi = jnp.where(m, 1, 0)
            pos = jnp.where(m, q * NSLOT + cnts[q] + plsc.cumsum(mi) - 1,
                            QN + lanes)
            plsc.store_scatter(idxl, [pos], idxs)
            out.append(cnts[q] + plsc.all_reduce_population_count(m))
        return tuple(out)

    zero = jnp.zeros((16,), jnp.int32)
    cnts = lax.fori_loop(0, NCHUNK, part_body, (zero,) * QMAX)
    ms = [jnp.max(c) for c in cnts]

    for q in range(QMAX):
        Q = q * NSLOT

        def box_body(p, ncl, Q=Q):
            psp = jnp.full((16,), p, jnp.int32)
            isp = plsc.load_gather(idxl, [Q + psp])
            ia = isp * 6
            b0 = plsc.load_gather(xv, [ia])
            b1 = plsc.load_gather(xv, [ia + 1])
            b2 = plsc.load_gather(xv, [ia + 2])
            b3 = plsc.load_gather(xv, [ia + 3])
            s = plsc.load_gather(xv, [ia + 4])
            a1 = (b2 - b0) * (b3 - b1)

            def ch_body(ch, jf, Q=Q):
                base = ch * 16
                m0 = mcoord[pl.ds(0 * QN + Q + base, 16)]
                m1 = mcoord[pl.ds(1 * QN + Q + base, 16)]
                m2 = mcoord[pl.ds(2 * QN + Q + base, 16)]
                m3 = mcoord[pl.ds(3 * QN + Q + base, 16)]
                iw = jnp.maximum(jnp.minimum(b2, m2) - jnp.maximum(b0, m0), 0.0)
                ih = jnp.maximum(jnp.minimum(b3, m3) - jnp.maximum(b1, m1), 0.0)
                inter = iw * ih
                a2 = (m2 - m0) * (m3 - m1)
                iou = inter / (a1 + a2 - inter)
                hit = (iou > IOU_T) & ((lanes + base) < ncl)
                pc = plsc.all_reduce_population_count(hit)
                ff = plsc.all_reduce_ffs(hit)
                return jnp.where((jf >= 16384) & (pc > 0), base + ff, jf)

            nch = (ncl + 15) >> 4
            jf = lax.fori_loop(0, nch, ch_body,
                               jnp.full((16,), 16384, jnp.int32))
            anyv = jf < 16384
            anyh = jnp.max(jnp.where(anyv, 1, 0)) > 0
            ksp = jnp.where(anyv, jf, jnp.full((16,), ncl, jnp.int32))
            addr = Q + ksp
            old_ss = plsc.load_gather(sacc, [addr])
            ssc = jnp.where(anyv, old_ss + s, s)
            plsc.store_scatter(sacc, [addr], ssc)
            old_c = plsc.load_gather(cacc, [addr])
            plsc.store_scatter(cacc, [addr],
                               jnp.where(anyv, old_c + 1.0, 1.0))
            for d, bd in enumerate((b0, b1, b2, b3)):
                old_w = plsc.load_gather(wacc, [d * QN + addr])
                wc = jnp.where(anyv, old_w + s * bd, s * bd)
                plsc.store_scatter(wacc, [d * QN + addr], wc)
                plsc.store_scatter(mcoord, [d * QN + addr], wc / ssc)
            crv = jnp.where(anyv, jnp.full((16,), -1, jnp.int32),
                            jnp.full((16,), ncl, jnp.int32))
            plsc.store_scatter(created, [Q + psp], crv)
            return ncl + jnp.where(anyh, 0, 1)

        lax.fori_loop(0, ms[q], box_body, jnp.int32(0))

    nfired = jnp.int32(0)
    for q in range(QMAX):
        Q = q * NSLOT
        cfq = cfs[q]

        def out_body(p, apos, Q=Q, cfq=cfq):
            psp = jnp.full((16,), p, jnp.int32)
            isp = plsc.load_gather(idxl, [Q + psp])
            ksp = plsc.load_gather(created, [Q + psp])
            isnew = ksp >= 0
            addr = Q + jnp.maximum(ksp, 0)
            ssv = plsc.load_gather(sacc, [addr])
            cnv = plsc.load_gather(cacc, [addr])
            scv = ssv / cnv
            keyv = jnp.where(isnew, scv, jnp.full((16,), NEG, jnp.float32))
            tbv = cfq * 1024.0 + isp.astype(jnp.float32)
            ab = apos * 16
            absp = jnp.full((16,), ab, jnp.int32)
            for d in range(4):
                wv = plsc.load_gather(wacc, [d * QN + addr])
                plsc.store_scatter(arena, [absp + d], wv / ssv)
            plsc.store_scatter(arena, [absp + 4], scv)
            plsc.store_scatter(arena, [absp + 5],
                               jnp.full((16,), cfq, jnp.float32))
            plsc.store_scatter(arena, [absp + 6], keyv)
            plsc.store_scatter(arena, [absp + 7], tbv)
            i_sc = jnp.max(isp)
            pltpu.async_copy(arena.at[pl.ds(ab, 16)],
                             buf_hbm.at[pl.ds(i_sc * 16, 16)], sem)
            return apos + 1

        nfired = lax.fori_loop(0, ms[q], out_body, nfired)

    def drain_body(p, c):
        pltpu.make_async_copy(x_hbm.at[pl.ds(0, 16)], rowv, sem).wait()
        return c

    lax.fori_loop(0, nfired, drain_body, jnp.int32(0))


def _rank_tc_body(buf_ref, o_ref):
    jcol = lax.broadcasted_iota(jnp.int32, (NSLOT, 1), 0)
    vcol = jcol < PRE
    key_c = jnp.where(vcol, buf_ref[:, 6:7], NEG)
    tb_c = jnp.where(vcol, buf_ref[:, 7:8], 2.0e8 + jcol.astype(jnp.float32))
    key_r = jnp.reshape(key_c, (1, NSLOT))
    tb_r = jnp.reshape(tb_c, (1, NSLOT))
    before = (key_c > key_r) | ((key_c == key_r) & (tb_c < tb_r))
    rank = jnp.sum(before.astype(jnp.float32), axis=0, keepdims=True)
    rsel = lax.broadcasted_iota(jnp.int32, (OUTN, 1), 0).astype(jnp.float32)
    onehot = (rank == rsel).astype(jnp.float32)
    bufc = jnp.where(vcol, buf_ref[:], 0.0)
    res = lax.dot_general(
        onehot, bufc, (((1,), (0,)), ((), ())),
        precision=lax.Precision.HIGHEST,
        preferred_element_type=jnp.float32)
    o_ref[:] = res[:POST, :6]


@jax.jit
def kernel(x):
    xflat = x.astype(jnp.float32).reshape(x.shape[0] * 6)

    mesh = plsc.VectorSubcoreMesh(core_axis_name="c", subcore_axis_name="s",
                                  num_cores=NC, num_subcores=NS)
    phase1 = pl.kernel(
        _wbf_sc_body,
        out_type=jax.ShapeDtypeStruct((NSLOT * 16,), jnp.float32),
        mesh=mesh,
        compiler_params=pltpu.CompilerParams(needs_layout_passes=False),
        scratch_types=[
            pltpu.VMEM((7200,), jnp.float32),
            pltpu.VMEM((QN + 16,), jnp.int32),
            pltpu.VMEM((QN,), jnp.int32),
            pltpu.VMEM((4 * QN,), jnp.float32),
            pltpu.VMEM((4 * QN,), jnp.float32),
            pltpu.VMEM((QN,), jnp.float32),
            pltpu.VMEM((QN,), jnp.float32),
            pltpu.VMEM((NSLOT * 16,), jnp.float32),
            pltpu.VMEM((16,), jnp.float32),
            pltpu.SemaphoreType.DMA,
        ],
    )
    buf = phase1(xflat).reshape(NSLOT, 16)

    return pl.pallas_call(
        _rank_tc_body,
        out_shape=jax.ShapeDtypeStruct((POST, 6), jnp.float32),
    )(buf)


# speedup vs baseline: 1.0432x; 1.0217x over previous
import functools

import jax
import jax.numpy as jnp
from jax import lax
from jax.experimental import pallas as pl
from jax.experimental.pallas import tpu as pltpu
from jax.experimental.pallas import tpu_sc as plsc

PRE = 1000
IOU_T = 0.55
POST = 300
NSLOT = 1024
NCHUNK = NSLOT // 16
NC, NS = 2, 16
NW = NC * NS
NCLS = 80
QMAX = -(-NCLS // NW)
QN = QMAX * NSLOT
OUTN = 304
NEG = -3.0e38


def _wbf_sc_body(x_hbm, buf_hbm, xv, idxl, created, mcoord, wacc, sacc,
                 cacc, arena, rowv, sem):
    wid = lax.axis_index("s") * NC + lax.axis_index("c")
    lanes = lax.broadcasted_iota(jnp.int32, (16,), 0)

    pltpu.sync_copy(x_hbm, xv)

    cfs = [(wid + NW * q).astype(jnp.float32) for q in range(QMAX)]

    def part_body(ch, cnts):
        idxs = lanes + ch * 16
        cls_chunk = plsc.load_gather(xv, [idxs * 6 + 5])
        live = idxs < PRE
        out = []
        for q in range(QMAX):
            m = (cls_chunk == cfs[q]) & live
            mi = jnp.where(m, 1, 0)
            pos = jnp.where(m, q * NSLOT + cnts[q] + plsc.cumsum(mi) - 1,
                            QN + lanes)
            plsc.store_scatter(idxl, [pos], idxs)
            out.append(cnts[q] + plsc.all_reduce_population_count(m))
        return tuple(out)

    zero = jnp.zeros((16,), jnp.int32)
    cnts = lax.fori_loop(0, NCHUNK, part_body, (zero,) * QMAX)
    ms = [jnp.max(c) for c in cnts]

    for q in range(QMAX):
        Q = q * NSLOT

        def box_body(p, ncl, Q=Q):
            psp = jnp.full((16,), p, jnp.int32)
            isp = plsc.load_gather(idxl, [Q + psp])
            ia = isp * 6
            b0 = plsc.load_gather(xv, [ia])
            b1 = plsc.load_gather(xv, [ia + 1])
            b2 = plsc.load_gather(xv, [ia + 2])
            b3 = plsc.load_gather(xv, [ia + 3])
            s = plsc.load_gather(xv, [ia + 4])
            a1 = (b2 - b0) * (b3 - b1)

            def ch_body(ch, jf, Q=Q):
                base = ch * 16
                m0 = mcoord[pl.ds(0 * QN + Q + base, 16)]
                m1 = mcoord[pl.ds(1 * QN + Q + base, 16)]
                m2 = mcoord[pl.ds(2 * QN + Q + base, 16)]
                m3 = mcoord[pl.ds(3 * QN + Q + base, 16)]
                iw = jnp.maximum(jnp.minimum(b2, m2) - jnp.maximum(b0, m0), 0.0)
                ih = jnp.maximum(jnp.minimum(b3, m3) - jnp.maximum(b1, m1), 0.0)
                inter = iw * ih
                a2 = (m2 - m0) * (m3 - m1)
                iou = inter / (a1 + a2 - inter)
                hit = (iou > IOU_T) & ((lanes + base) < ncl)
                pc = plsc.all_reduce_population_count(hit)
                ff = plsc.all_reduce_ffs(hit)
                return jnp.where((jf >= 16384) & (pc > 0), base + ff, jf)

            nch = (ncl + 15) >> 4
            jf = lax.fori_loop(0, nch, ch_body,
                               jnp.full((16,), 16384, jnp.int32))
            anyv = jf < 16384
            anyh = jnp.max(jnp.where(anyv, 1, 0)) > 0
            ksp = jnp.where(anyv, jf, jnp.full((16,), ncl, jnp.int32))
            addr = Q + ksp
            old_ss = plsc.load_gather(sacc, [addr])
            ssc = jnp.where(anyv, old_ss + s, s)
            plsc.store_scatter(sacc, [addr], ssc)
            old_c = plsc.load_gather(cacc, [addr])
            plsc.store_scatter(cacc, [addr],
                               jnp.where(anyv, old_c + 1.0, 1.0))
            for d, bd in enumerate((b0, b1, b2, b3)):
                old_w = plsc.load_gather(wacc, [d * QN + addr])
                wc = jnp.where(anyv, old_w + s * bd, s * bd)
                plsc.store_scatter(wacc, [d * QN + addr], wc)
                plsc.store_scatter(mcoord, [d * QN + addr], wc / ssc)
            crv = jnp.where(anyv, jnp.full((16,), -1, jnp.int32),
                            jnp.full((16,), ncl, jnp.int32))
            plsc.store_scatter(created, [Q + psp], crv)
            return ncl + jnp.where(anyh, 0, 1)

        lax.fori_loop(0, ms[q], box_body, jnp.int32(0))

    nfired = jnp.int32(0)
    for q in range(QMAX):
        Q = q * NSLOT
        cfq = cfs[q]

        def out_body(p, apos, Q=Q, cfq=cfq):
            psp = jnp.full((16,), p, jnp.int32)
            isp = plsc.load_gather(idxl, [Q + psp])
            ksp = plsc.load_gather(created, [Q + psp])
            isnew = ksp >= 0
            addr = Q + jnp.maximum(ksp, 0)
            w0 = plsc.load_gather(wacc, [0 * QN + addr])
            w1 = plsc.load_gather(wacc, [1 * QN + addr])
            w2 = plsc.load_gather(wacc, [2 * QN + addr])
            w3 = plsc.load_gather(wacc, [3 * QN + addr])
            ssv = plsc.load_gather(sacc, [addr])
            cnv = plsc.load_gather(cacc, [addr])
            tbv = cfq * 1024.0 + isp.astype(jnp.float32)
            keyn = jnp.where(isnew, ssv, NEG)
            keyd = jnp.where(isnew, cnv, 1.0)
            num = jnp.zeros((16,), jnp.float32)
            den = jnp.ones((16,), jnp.float32)
            for li, (nv, dv) in enumerate((
                    (w0, ssv), (w1, ssv), (w2, ssv), (w3, ssv),
                    (ssv, cnv), (cfq, 1.0), (keyn, keyd), (tbv, 1.0))):
                num = jnp.where(lanes == li, nv, num)
                den = jnp.where(lanes == li, dv, den)
            ab = apos * 16
            arena[pl.ds(ab, 16)] = num / den
            i_sc = jnp.max(isp)
            pltpu.async_copy(arena.at[pl.ds(ab, 16)],
                             buf_hbm.at[pl.ds(i_sc * 16, 16)], sem)
            return apos + 1

        nfired = lax.fori_loop(0, ms[q], out_body, nfired)

    def drain_body(p, c):
        pltpu.make_async_copy(x_hbm.at[pl.ds(0, 16)], rowv, sem).wait()
        return c

    lax.fori_loop(0, nfired, drain_body, jnp.int32(0))


def _rank_tc_body(buf_ref, o_ref):
    jcol = lax.broadcasted_iota(jnp.int32, (NSLOT, 1), 0)
    vcol = jcol < PRE
    key_c = jnp.where(vcol, buf_ref[:, 6:7], NEG)
    tb_c = jnp.where(vcol, buf_ref[:, 7:8], 2.0e8 + jcol.astype(jnp.float32))
    key_r = jnp.reshape(key_c, (1, NSLOT))
    tb_r = jnp.reshape(tb_c, (1, NSLOT))
    before = (key_c > key_r) | ((key_c == key_r) & (tb_c < tb_r))
    rank = jnp.sum(before.astype(jnp.float32), axis=0, keepdims=True)
    rsel = lax.broadcasted_iota(jnp.int32, (OUTN, 1), 0).astype(jnp.float32)
    onehot = (rank == rsel).astype(jnp.float32)
    bufc = jnp.where(vcol, buf_ref[:], 0.0)
    res = lax.dot_general(
        onehot, bufc, (((1,), (0,)), ((), ())),
        precision=lax.Precision.HIGHEST,
        preferred_element_type=jnp.float32)
    o_ref[:] = res[:POST, :6]


@jax.jit
def kernel(x):
    xflat = x.astype(jnp.float32).reshape(x.shape[0] * 6)

    mesh = plsc.VectorSubcoreMesh(core_axis_name="c", subcore_axis_name="s",
                                  num_cores=NC, num_subcores=NS)
    phase1 = pl.kernel(
        _wbf_sc_body,
        out_type=jax.ShapeDtypeStruct((NSLOT * 16,), jnp.float32),
        mesh=mesh,
        compiler_params=pltpu.CompilerParams(needs_layout_passes=False),
        scratch_types=[
            pltpu.VMEM((7200,), jnp.float32),
            pltpu.VMEM((QN + 16,), jnp.int32),
            pltpu.VMEM((QN,), jnp.int32),
            pltpu.VMEM((4 * QN,), jnp.float32),
            pltpu.VMEM((4 * QN,), jnp.float32),
            pltpu.VMEM((QN,), jnp.float32),
            pltpu.VMEM((QN,), jnp.float32),
            pltpu.VMEM((NSLOT * 16,), jnp.float32),
            pltpu.VMEM((16,), jnp.float32),
            pltpu.SemaphoreType.DMA,
        ],
    )
    buf = phase1(xflat).reshape(NSLOT, 16)

    return pl.pallas_call(
        _rank_tc_body,
        out_shape=jax.ShapeDtypeStruct((POST, 6), jnp.float32),
    )(buf)
